# Initial kernel scaffold; baseline (speedup 1.0000x reference)
import jax
import jax.numpy as jnp
from jax.experimental import pallas as pl


def kernel(pred, label):
    raise NotImplementedError("write your pallas kernel here")


# speedup vs baseline: 2.8842x; 2.8842x over previous
import functools

import jax
import jax.numpy as jnp
from jax import lax
from jax.experimental import pallas as pl
from jax.experimental.pallas import tpu as pltpu

_LS = 0.1
_BLK_W = 2048


def _pass_body(c_total, label_ref, pred_ref, out_ref, s_acc, m_acc, g_acc, z_acc):
    j = pl.program_id(0)
    nblk = pl.num_programs(0)
    blk_b, blk_w = pred_ref.shape

    @pl.when(j == 0)
    def _init():
        s_acc[...] = jnp.zeros_like(s_acc)
        m_acc[...] = jnp.full_like(m_acc, -jnp.inf)
        g_acc[...] = jnp.zeros_like(g_acc)
        z_acc[...] = jnp.zeros_like(z_acc)

    x = pred_ref[...]
    col = j * blk_w + lax.broadcasted_iota(jnp.int32, (blk_b, blk_w), 1)
    valid = col < c_total
    xm = jnp.where(valid, x, -jnp.inf)
    match = col == label_ref[...]
    s_acc[...] += jnp.sum(jnp.exp(xm), axis=1, keepdims=True)
    m_acc[...] = jnp.maximum(m_acc[...], jnp.max(xm, axis=1, keepdims=True))
    g_acc[...] += jnp.sum(jnp.where(match, x, 0.0), axis=1, keepdims=True)
    z_acc[...] += jnp.sum(
        jnp.where(valid & (x == 0.0) & jnp.logical_not(match), 1.0, 0.0),
        axis=1, keepdims=True)

    @pl.when(j == nblk - 1)
    def _finish():
        s = s_acc[...]
        m = m_acc[...]
        g = g_acc[...]
        masked = jnp.exp(m) > 0.5 * s
        at_label = masked & (g == m)
        base = -(1.0 - _LS) * jnp.log(jnp.where(at_label, 1.0 - g, g))
        extra = jnp.where(masked & jnp.logical_not(at_label),
                          -_LS * jnp.log(1.0 - m), 0.0)
        loss = jnp.mean(base + extra)
        has_nan = jnp.max(z_acc[...]) > 0.0
        out_ref[...] = jnp.full((1, 1), jnp.where(has_nan, jnp.float32(jnp.nan), loss))


@functools.partial(jax.jit, static_argnames=("interpret",))
def kernel(pred, label, interpret=False):
    b, c = pred.shape
    blk_w = min(_BLK_W, c)
    nblk = pl.cdiv(c, blk_w)
    label2d = label.reshape(b, 1).astype(jnp.int32)
    out = pl.pallas_call(
        functools.partial(_pass_body, c),
        grid=(nblk,),
        in_specs=[
            pl.BlockSpec((b, 1), lambda j: (0, 0)),
            pl.BlockSpec((b, blk_w), lambda j: (0, j)),
        ],
        out_specs=pl.BlockSpec((1, 1), lambda j: (0, 0)),
        out_shape=jax.ShapeDtypeStruct((1, 1), jnp.float32),
        scratch_shapes=[
            pltpu.VMEM((b, 1), jnp.float32),
            pltpu.VMEM((b, 1), jnp.float32),
            pltpu.VMEM((b, 1), jnp.float32),
            pltpu.VMEM((b, 1), jnp.float32),
        ],
        interpret=interpret,
    )(label2d, pred)
    return out.reshape(())
